# baseline (speedup 1.0000x reference)
import functools

import jax
import jax.numpy as jnp
from jax import lax
from jax.experimental import pallas as pl
from jax.experimental.pallas import tpu as pltpu
from jax.experimental.pallas import tpu_sc as plsc

D = 768
B = 4
S = 2048
TOKENS = B * S
EPS = 1e-5

NC, NS = 2, 16
NW = NC * NS
PER_W = S // NW
CH = 32
NCH = PER_W // CH


@functools.lru_cache(maxsize=1)
def _make_gather_rows():
    mesh = plsc.VectorSubcoreMesh(
        core_axis_name="c", subcore_axis_name="s", num_cores=NC, num_subcores=NS
    )

    @functools.partial(
        pl.kernel,
        mesh=mesh,
        out_type=jax.ShapeDtypeStruct((S, D), jnp.float32),
        scratch_types=[
            pltpu.VMEM((NCH, CH), jnp.int32),
            pltpu.VMEM((CH, D), jnp.float32),
            pltpu.VMEM((CH, D), jnp.float32),
            pltpu.SemaphoreType.DMA,
            pltpu.SemaphoreType.DMA,
            pltpu.SemaphoreType.DMA,
            pltpu.SemaphoreType.DMA,
        ],
    )
    def _gather_rows(
        ids_hbm, word_hbm, out_hbm, idx_v, buf0, buf1, gsem0, gsem1, ssem0, ssem1
    ):
        wid = lax.axis_index("s") * NC + lax.axis_index("c")
        base = wid * PER_W
        for i in range(NCH):
            pltpu.sync_copy(ids_hbm.at[0, pl.ds(base + i * CH, CH)], idx_v.at[i])
        bufs = (buf0, buf1)
        gsems = (gsem0, gsem1)
        ssems = (ssem0, ssem1)
        gathers = [
            pltpu.async_copy(word_hbm.at[idx_v.at[i]], bufs[i], gsems[i])
            for i in range(NCH)
        ]
        stores = [None] * NCH
        for i in range(NCH):
            gathers[i].wait()
            stores[i] = pltpu.async_copy(
                bufs[i], out_hbm.at[pl.ds(base + i * CH, CH)], ssems[i]
            )
        for i in range(NCH):
            stores[i].wait()

    return _gather_rows


ROWS_BLK = 1024
POS_BLKS = S // ROWS_BLK


def _ln_body(g_ref, pos_ref, tt_ref, type_ref, gamma_ref, beta_ref, _prev_ref, o_ref):
    x = g_ref[...] + pos_ref[...]
    tt = tt_ref[0, :, 0:1].astype(jnp.float32)
    t0 = type_ref[0:1, :]
    t1 = type_ref[1:2, :]
    x = x + t0 + tt * (t1 - t0)
    mu = jnp.mean(x, axis=1, keepdims=True)
    xc = x - mu
    var = jnp.mean(xc * xc, axis=1, keepdims=True)
    y = xc * lax.rsqrt(var + EPS)
    o_ref[0] = y * gamma_ref[...] + beta_ref[...]


def _make_ln_call(row, alias):
    return pl.pallas_call(
        _ln_body,
        grid=(POS_BLKS,),
        in_specs=[
            pl.BlockSpec((ROWS_BLK, D), lambda sb: (sb, 0)),
            pl.BlockSpec((ROWS_BLK, D), lambda sb: (sb, 0)),
            pl.BlockSpec((1, ROWS_BLK, 1), lambda sb: (0, sb, 0)),
            pl.BlockSpec((2, D), lambda sb: (0, 0)),
            pl.BlockSpec((1, D), lambda sb: (0, 0)),
            pl.BlockSpec((1, D), lambda sb: (0, 0)),
            pl.BlockSpec(memory_space=pl.ANY),
        ],
        out_specs=pl.BlockSpec((1, ROWS_BLK, D), lambda sb: (row, sb, 0)),
        out_shape=jax.ShapeDtypeStruct((B, S, D), jnp.float32),
        input_output_aliases={6: 0} if alias else {},
    )


_ln_calls = tuple(_make_ln_call(r, r > 0) for r in range(B))


def kernel(input_ids, token_type_ids, word_emb, pos_emb, type_emb, gamma, beta):
    ids = input_ids.astype(jnp.int32)
    tt = token_type_ids.astype(jnp.int32)[:, :, None]
    gamma2 = gamma.reshape(1, D)
    beta2 = beta.reshape(1, D)
    sc = _make_gather_rows()
    gs = [sc(ids[r : r + 1], word_emb) for r in range(B)]
    out = jnp.zeros((1,), jnp.float32)
    for r in range(B):
        out = _ln_calls[r](
            gs[r], pos_emb, tt[r : r + 1], type_emb, gamma2, beta2, out
        )
    return out


# speedup vs baseline: 1.0746x; 1.0746x over previous
import functools

import jax
import jax.numpy as jnp
from jax import lax
from jax.experimental import pallas as pl
from jax.experimental.pallas import tpu as pltpu
from jax.experimental.pallas import tpu_sc as plsc

D = 768
B = 4
S = 2048
TOKENS = B * S
EPS = 1e-5

SH = S // 2
TOK_H = B * SH

NC, NS = 2, 16
NW = NC * NS
PER_W = TOK_H // NW
W_PER_B = SH // PER_W
CH = 32
NCH = PER_W // CH
NBUF = 3


@functools.lru_cache(maxsize=1)
def _make_gather_rows():
    mesh = plsc.VectorSubcoreMesh(
        core_axis_name="c", subcore_axis_name="s", num_cores=NC, num_subcores=NS
    )

    @functools.partial(
        pl.kernel,
        mesh=mesh,
        out_type=jax.ShapeDtypeStruct((TOK_H, D), jnp.float32),
        scratch_types=[
            pltpu.VMEM((NCH, CH), jnp.int32),
            *[pltpu.VMEM((CH, D), jnp.float32) for _ in range(NBUF)],
            *[pltpu.SemaphoreType.DMA for _ in range(2 * NBUF)],
        ],
    )
    def _gather_rows(ids_hbm, word_hbm, out_hbm, idx_v, *bufs_sems):
        bufs = bufs_sems[:NBUF]
        gsems = bufs_sems[NBUF : 2 * NBUF]
        ssems = bufs_sems[2 * NBUF :]
        wid = lax.axis_index("s") * NC + lax.axis_index("c")
        b = wid // W_PER_B
        s0 = (wid % W_PER_B) * PER_W
        base = wid * PER_W
        for i in range(NCH):
            pltpu.sync_copy(ids_hbm.at[b, pl.ds(s0 + i * CH, CH)], idx_v.at[i])

        def gather(i):
            return pltpu.async_copy(
                word_hbm.at[idx_v.at[i]], bufs[i % NBUF], gsems[i % NBUF]
            )

        def store(i):
            return pltpu.async_copy(
                bufs[i % NBUF], out_hbm.at[pl.ds(base + i * CH, CH)], ssems[i % NBUF]
            )

        gathers = [None] * NCH
        stores = [None] * NCH
        waited = [False] * NCH
        gathers[0] = gather(0)
        for i in range(NCH):
            gathers[i].wait()
            stores[i] = store(i)
            j = i + 1
            if j < NCH:
                if j >= NBUF:
                    stores[j - NBUF].wait()
                    waited[j - NBUF] = True
                gathers[j] = gather(j)
        for i in range(NCH):
            if not waited[i]:
                stores[i].wait()

    return _gather_rows


ROWS_BLK = 1024


def _ln_body(g_ref, pos_ref, tt_ref, type_ref, gamma_ref, beta_ref, _prev_ref, o_ref):
    x = g_ref[...] + pos_ref[...]
    tt = tt_ref[0, :, 0:1].astype(jnp.float32)
    t0 = type_ref[0:1, :]
    t1 = type_ref[1:2, :]
    x = x + t0 + tt * (t1 - t0)
    mu = jnp.mean(x, axis=1, keepdims=True)
    xc = x - mu
    var = jnp.mean(xc * xc, axis=1, keepdims=True)
    y = xc * lax.rsqrt(var + EPS)
    o_ref[0] = y * gamma_ref[...] + beta_ref[...]


def _make_ln_call(half, alias):
    return pl.pallas_call(
        _ln_body,
        grid=(B,),
        in_specs=[
            pl.BlockSpec((ROWS_BLK, D), lambda b: (b, 0)),
            pl.BlockSpec((ROWS_BLK, D), lambda b: (half, 0)),
            pl.BlockSpec((1, ROWS_BLK, 1), lambda b: (b, half, 0)),
            pl.BlockSpec((2, D), lambda b: (0, 0)),
            pl.BlockSpec((1, D), lambda b: (0, 0)),
            pl.BlockSpec((1, D), lambda b: (0, 0)),
            pl.BlockSpec(memory_space=pl.ANY),
        ],
        out_specs=pl.BlockSpec((1, ROWS_BLK, D), lambda b: (b, half, 0)),
        out_shape=jax.ShapeDtypeStruct((B, S, D), jnp.float32),
        input_output_aliases={6: 0} if alias else {},
    )


_ln_calls = (_make_ln_call(0, False), _make_ln_call(1, True))


def kernel(input_ids, token_type_ids, word_emb, pos_emb, type_emb, gamma, beta):
    ids = input_ids.astype(jnp.int32)
    tt = token_type_ids.astype(jnp.int32)[:, :, None]
    gamma2 = gamma.reshape(1, D)
    beta2 = beta.reshape(1, D)
    sc = _make_gather_rows()
    g0 = sc(ids[:, :SH], word_emb)
    g1 = sc(ids[:, SH:], word_emb)
    dummy = jnp.zeros((1,), jnp.float32)
    out = _ln_calls[0](g0, pos_emb, tt, type_emb, gamma2, beta2, dummy)
    out = _ln_calls[1](g1, pos_emb, tt, type_emb, gamma2, beta2, out)
    return out


# speedup vs baseline: 1.1164x; 1.0390x over previous
import functools

import jax
import jax.numpy as jnp
from jax import lax
from jax.experimental import pallas as pl
from jax.experimental.pallas import tpu as pltpu
from jax.experimental.pallas import tpu_sc as plsc

D = 768
B = 4
S = 2048
TOKENS = B * S
EPS = 1e-5

SH = S // 2
TOK_H = B * SH

NC, NS = 2, 16
NW = NC * NS
PER_W = TOK_H // NW
W_PER_B = SH // PER_W
CH = 64
NCH = PER_W // CH
NBUF = 2


@functools.lru_cache(maxsize=1)
def _make_gather_rows():
    mesh = plsc.VectorSubcoreMesh(
        core_axis_name="c", subcore_axis_name="s", num_cores=NC, num_subcores=NS
    )

    @functools.partial(
        pl.kernel,
        mesh=mesh,
        out_type=jax.ShapeDtypeStruct((TOK_H, D), jnp.float32),
        scratch_types=[
            pltpu.VMEM((NCH, CH), jnp.int32),
            *[pltpu.VMEM((CH, D), jnp.float32) for _ in range(NBUF)],
            *[pltpu.SemaphoreType.DMA for _ in range(2 * NBUF)],
        ],
    )
    def _gather_rows(ids_hbm, word_hbm, out_hbm, idx_v, *bufs_sems):
        bufs = bufs_sems[:NBUF]
        gsems = bufs_sems[NBUF : 2 * NBUF]
        ssems = bufs_sems[2 * NBUF :]
        wid = lax.axis_index("s") * NC + lax.axis_index("c")
        b = wid // W_PER_B
        s0 = (wid % W_PER_B) * PER_W
        base = wid * PER_W
        for i in range(NCH):
            pltpu.sync_copy(ids_hbm.at[b, pl.ds(s0 + i * CH, CH)], idx_v.at[i])

        def gather(i):
            return pltpu.async_copy(
                word_hbm.at[idx_v.at[i]], bufs[i % NBUF], gsems[i % NBUF]
            )

        def store(i):
            return pltpu.async_copy(
                bufs[i % NBUF], out_hbm.at[pl.ds(base + i * CH, CH)], ssems[i % NBUF]
            )

        gathers = [gather(i) for i in range(NCH)]
        stores = [None] * NCH
        for i in range(NCH):
            gathers[i].wait()
            stores[i] = store(i)
        for i in range(NCH):
            stores[i].wait()

    return _gather_rows


ROWS_BLK = 1024


def _ln_body(g_ref, pos_ref, tt_ref, type_ref, gamma_ref, beta_ref, _prev_ref, o_ref):
    x = g_ref[...] + pos_ref[...]
    tt = tt_ref[0, :, 0:1].astype(jnp.float32)
    t0 = type_ref[0:1, :]
    t1 = type_ref[1:2, :]
    x = x + t0 + tt * (t1 - t0)
    mu = jnp.mean(x, axis=1, keepdims=True)
    xc = x - mu
    var = jnp.mean(xc * xc, axis=1, keepdims=True)
    y = xc * lax.rsqrt(var + EPS)
    o_ref[0] = y * gamma_ref[...] + beta_ref[...]


def _make_ln_call(half, alias):
    return pl.pallas_call(
        _ln_body,
        grid=(B,),
        in_specs=[
            pl.BlockSpec((ROWS_BLK, D), lambda b: (b, 0)),
            pl.BlockSpec((ROWS_BLK, D), lambda b: (half, 0)),
            pl.BlockSpec((1, ROWS_BLK, 1), lambda b: (b, half, 0)),
            pl.BlockSpec((2, D), lambda b: (0, 0)),
            pl.BlockSpec((1, D), lambda b: (0, 0)),
            pl.BlockSpec((1, D), lambda b: (0, 0)),
            pl.BlockSpec(memory_space=pl.ANY),
        ],
        out_specs=pl.BlockSpec((1, ROWS_BLK, D), lambda b: (b, half, 0)),
        out_shape=jax.ShapeDtypeStruct((B, S, D), jnp.float32),
        input_output_aliases={6: 0} if alias else {},
    )


_ln_calls = (_make_ln_call(0, False), _make_ln_call(1, True))


def kernel(input_ids, token_type_ids, word_emb, pos_emb, type_emb, gamma, beta):
    ids = input_ids.astype(jnp.int32)
    tt = token_type_ids.astype(jnp.int32)[:, :, None]
    gamma2 = gamma.reshape(1, D)
    beta2 = beta.reshape(1, D)
    sc = _make_gather_rows()
    g0 = sc(ids[:, :SH], word_emb)
    g1 = sc(ids[:, SH:], word_emb)
    dummy = jnp.zeros((1,), jnp.float32)
    out = _ln_calls[0](g0, pos_emb, tt, type_emb, gamma2, beta2, dummy)
    out = _ln_calls[1](g1, pos_emb, tt, type_emb, gamma2, beta2, out)
    return out
